# Initial kernel scaffold; baseline (speedup 1.0000x reference)
import jax
import jax.numpy as jnp
from jax.experimental import pallas as pl


def kernel(x, in_degree, out_degree, atom_table, in_table, out_table, graph_token):
    raise NotImplementedError("write your pallas kernel here")


# speedup vs baseline: 2.4973x; 2.4973x over previous
import functools

import jax
import jax.numpy as jnp
from jax import lax
from jax.experimental import pallas as pl
from jax.experimental.pallas import tpu as pltpu
from jax.experimental.pallas import tpu_sc as plsc

L = 16
NC = 2
NS = 16
NW = NC * NS
R = 11
OC = 16
GPB = R
NBUF = 4


def _build_sc_kernel(B, N, H, NA, NI, NO):
    ZERO_ROW = NA + NI + NO
    IPB = N * R
    NG = IPB // L
    NBLK = N // OC
    B_PER_W = B // NW
    NV = H // L
    NOUT = N + 1

    mesh = plsc.VectorSubcoreMesh(core_axis_name="c", subcore_axis_name="s")

    @functools.partial(
        pl.kernel,
        out_type=jax.ShapeDtypeStruct((B * NOUT, H), jnp.float32),
        mesh=mesh,
        scratch_types=[
            pltpu.VMEM((IPB,), jnp.int32),
            pltpu.VMEM((IPB,), jnp.int32),
            [pltpu.VMEM((L,), jnp.int32) for _ in range(NBUF)],
            [pltpu.VMEM((L, H), jnp.float32) for _ in range(NBUF)],
            [pltpu.VMEM((OC, H), jnp.float32) for _ in range(2)],
            [pltpu.VMEM((L,), jnp.int32) for _ in range(2)],
            pltpu.VMEM((L, H), jnp.float32),
            pltpu.VMEM((L,), jnp.int32),
            pltpu.VMEM((L,), jnp.int32),
            [pltpu.SemaphoreType.DMA for _ in range(NBUF)],
            [pltpu.SemaphoreType.DMA for _ in range(2)],
            pltpu.SemaphoreType.DMA,
        ],
    )
    def gnf(table_hbm, idx_hbm, gtok_hbm, out_hbm,
            raw_v, rm_v, idx_bufs, g_bufs, o_bufs, oi_bufs,
            gtok_v, tzi_v, ti_v, sems, osems, tsem):
        wid = lax.axis_index("s") * NC + lax.axis_index("c")
        lanes = lax.iota(jnp.int32, L)

        tzi_v[...] = lax.min(lanes, 0)
        pltpu.async_copy(gtok_hbm.at[tzi_v], gtok_v, tsem).wait()
        tok_rows = lax.min(wid * B_PER_W + lanes, B - 1) * NOUT
        ti_v[...] = tok_rows
        pltpu.async_copy(gtok_v, out_hbm.at[ti_v], tsem).wait()

        def fire(g):
            q = g % NBUF
            idx_bufs[q][...] = rm_v[pl.ds(L * g, L)]
            pltpu.async_copy(table_hbm.at[idx_bufs[q]], g_bufs[q], sems[q])

        def accumulate(g):
            q = g % NBUF
            gb = g_bufs[q]
            pltpu.make_async_copy(
                table_hbm.at[idx_bufs[q]], gb, sems[q]).wait()
            blk = (L * g) // (R * OC)
            o = o_bufs[blk % 2]

            def vbody(v, _):
                col = pl.ds(v * L, L)
                for i in range(L):
                    row = L * g + i
                    j = row // R - blk * OC
                    val = gb[i, col]
                    if row % R == 0:
                        o[j, col] = val
                    else:
                        plsc.addupdate(o.at[j, col], val)
                return 0

            lax.fori_loop(0, NV, vbody, 0, unroll=False)

        def batch_body(bi, _):
            b = wid * B_PER_W + bi
            pltpu.sync_copy(idx_hbm.at[pl.ds(b * IPB, IPB)], raw_v)
            for k in range(NG):
                rawv = raw_v[pl.ds(L * k, L)]
                t = lax.rem(lanes + (L * k), R)
                offv = jnp.where(t < 9, 0, jnp.where(t == 9, NA, NA + NI))
                rm_v[pl.ds(L * k, L)] = jnp.where(
                    rawv == 0, ZERO_ROW, rawv + offv)

            for g in range(NBUF - 1):
                fire(g)
            for g in range(NG):
                if g + NBUF - 1 < NG:
                    fire(g + NBUF - 1)
                blk = (L * g) // (R * OC)
                p = blk % 2
                if g == GPB * blk and blk >= 2:
                    pltpu.make_async_copy(
                        o_bufs[p], out_hbm.at[oi_bufs[p]], osems[p]).wait()
                accumulate(g)
                if g == GPB * (blk + 1) - 1:
                    rows = b * NOUT + 1 + OC * blk + lanes
                    oi_bufs[p][...] = rows
                    pltpu.async_copy(o_bufs[p], out_hbm.at[oi_bufs[p]],
                                     osems[p])
            for p in range(2):
                pltpu.make_async_copy(
                    o_bufs[p], out_hbm.at[oi_bufs[p]], osems[p]).wait()
            return 0

        lax.fori_loop(0, B_PER_W, batch_body, 0, unroll=False)

    return gnf


def kernel(x, in_degree, out_degree, atom_table, in_table, out_table,
           graph_token):
    B, N, F = x.shape
    H = atom_table.shape[1]
    NA, NI, NO = atom_table.shape[0], in_table.shape[0], out_table.shape[0]

    table = jnp.concatenate(
        [atom_table, in_table, out_table,
         jnp.zeros((1, H), atom_table.dtype)], axis=0)
    idx_all = jnp.concatenate(
        [x.astype(jnp.int32),
         in_degree[..., None].astype(jnp.int32),
         out_degree[..., None].astype(jnp.int32)], axis=-1).reshape(B * N * R)

    gnf = _build_sc_kernel(B, N, H, NA, NI, NO)
    out = gnf(table, idx_all, graph_token.astype(jnp.float32))
    return out.reshape(B, N + 1, H)
